# Initial kernel scaffold; baseline (speedup 1.0000x reference)
import jax
import jax.numpy as jnp
from jax.experimental import pallas as pl


def kernel(input_pos, k_val, v_val, k_cache, v_cache):
    raise NotImplementedError("write your pallas kernel here")


# speedup vs baseline: 1.0778x; 1.0778x over previous
import jax
import jax.numpy as jnp
from jax.experimental import pallas as pl
from jax.experimental.pallas import tpu as pltpu

MAX_B = 8
N_KV_HEAD = 8
MAX_SEQ = 4096
HEAD_DIM = 128
S = 16
BH = MAX_B * N_KV_HEAD


def _body(pos_ref, kc_ref, vc_ref, kv_ref, vv_ref, ko_ref, vo_ref):
    ko_ref[...] = kc_ref[...]
    vo_ref[...] = vc_ref[...]
    for i in range(S):
        p = pos_ref[i]
        ko_ref[0, pl.ds(p, 1), :] = kv_ref[0, pl.ds(i, 1), :]
        vo_ref[0, pl.ds(p, 1), :] = vv_ref[0, pl.ds(i, 1), :]


def kernel(input_pos, k_val, v_val, k_cache, v_cache):
    pos = input_pos.astype(jnp.int32)
    kc = k_cache.reshape(BH, MAX_SEQ, HEAD_DIM)
    vc = v_cache.reshape(BH, MAX_SEQ, HEAD_DIM)
    kv = k_val.reshape(BH, S, HEAD_DIM)
    vv = v_val.reshape(BH, S, HEAD_DIM)

    cache_spec = pl.BlockSpec((1, MAX_SEQ, HEAD_DIM), lambda i: (i, 0, 0))
    val_spec = pl.BlockSpec((1, S, HEAD_DIM), lambda i: (i, 0, 0))
    out_sds = jax.ShapeDtypeStruct((BH, MAX_SEQ, HEAD_DIM), jnp.float32)

    ko, vo = pl.pallas_call(
        _body,
        grid=(BH,),
        in_specs=[
            pl.BlockSpec(memory_space=pltpu.SMEM),
            cache_spec,
            cache_spec,
            val_spec,
            val_spec,
        ],
        out_specs=[cache_spec, cache_spec],
        out_shape=[out_sds, out_sds],
    )(pos, kc, vc, kv, vv)

    shape4 = (MAX_B, N_KV_HEAD, MAX_SEQ, HEAD_DIM)
    return (ko.reshape(shape4), vo.reshape(shape4))
